# baseline (speedup 1.0000x reference)
import functools

import jax
import jax.numpy as jnp
import numpy as np
from jax.experimental import pallas as pl
from jax.experimental.pallas import tpu as pltpu

FACTOR = 5
NEG_INF = float("-inf")


def _head_kernel(i_ref, q_ref, k_ref, v_ref, o_ref,
                 cnt_ref, m_ref, qr_ref, upd_ref, idx_ref,
                 *, u, blk_q, blk_c, d_head):
    L = q_ref.shape[0]
    D = d_head
    n_heads = q_ref.shape[1] // d_head
    L_K = cnt_ref.shape[1]
    scale = 1.0 / np.sqrt(D)

    @pl.when(pl.program_id(0) + pl.program_id(1) == 0)
    def _build_counts():
        kiota = jax.lax.broadcasted_iota(jnp.int32, (L, L_K), 1)
        acc = jnp.zeros((L, L_K), jnp.float32)
        for s in range(i_ref.shape[1]):
            acc = acc + (i_ref[:, s:s + 1] == kiota).astype(jnp.float32)
        cnt_ref[...] = acc

    tri = (jax.lax.broadcasted_iota(jnp.int32, (blk_c, blk_c), 0)
           >= jax.lax.broadcasted_iota(jnp.int32, (blk_c, blk_c), 1)
           ).astype(jnp.float32)
    n_cblk = L // blk_c
    carry = jnp.zeros((1, q_ref.shape[1]), jnp.float32)
    for b in range(n_cblk):
        rows = pl.ds(b * blk_c, blk_c)
        blk = jax.lax.dot_general(
            tri, v_ref[rows, :],
            (((1,), (0,)), ((), ())),
            preferred_element_type=jnp.float32,
            precision=jax.lax.Precision.HIGHEST) + carry
        o_ref[rows, :] = blk
        carry = blk[blk_c - 1:blk_c, :]

    lin = jax.lax.broadcasted_iota(jnp.int32, (1, L), 1)
    ri = jax.lax.broadcasted_iota(jnp.int32, (u, u), 0)
    ci = jax.lax.broadcasted_iota(jnp.int32, (u, u), 1)
    n_blk = L // blk_q
    for hh in range(n_heads):
        cols = slice(hh * D, (hh + 1) * D)

        for b in range(n_blk):
            rows = pl.ds(b * blk_q, blk_q)
            s_blk = jax.lax.dot_general(
                q_ref[rows, cols], k_ref[:, cols],
                (((1,), (1,)), ((), ())),
                preferred_element_type=jnp.float32,
                precision=jax.lax.Precision.DEFAULT)
            cnt = cnt_ref[rows, :]
            mx = jnp.max(jnp.where(cnt > 0.0, s_blk, NEG_INF), axis=1)
            sm = jnp.sum(s_blk * cnt, axis=1)
            m_ref[:, rows] = (mx - sm * (1.0 / L_K))[None, :]

        def topk_body(i, mv):
            mmax = jnp.max(mv)
            j = jnp.min(jnp.where(mv == mmax, lin, L))
            idx_ref[i] = j
            qr_ref[pl.ds(i, 1), :] = q_ref[pl.ds(j, 1), cols]
            return jnp.where(lin == j, NEG_INF, mv)

        jax.lax.fori_loop(0, u, topk_body, m_ref[...], unroll=False)

        s2 = jax.lax.dot_general(
            qr_ref[...], k_ref[0:u, cols],
            (((1,), (1,)), ((), ())),
            preferred_element_type=jnp.float32,
            precision=jax.lax.Precision.DEFAULT) * scale
        s2 = jnp.where(ci > ri, NEG_INF, s2)
        s2 = s2 - jnp.max(s2, axis=1, keepdims=True)
        e = jnp.exp(s2)
        attn = e / jnp.sum(e, axis=1, keepdims=True)
        upd_ref[...] = jax.lax.dot_general(
            attn, v_ref[0:u, cols],
            (((1,), (0,)), ((), ())),
            preferred_element_type=jnp.float32,
            precision=jax.lax.Precision.HIGHEST)

        def scat_body(i, _):
            j = idx_ref[i]
            o_ref[pl.ds(j, 1), cols] = upd_ref[pl.ds(i, 1), :]
            return 0

        jax.lax.fori_loop(0, u, scat_body, 0, unroll=False)


def kernel(queries, keys, values):
    B, L, H, D = queries.shape
    L_K = keys.shape[1]
    U_part = min(int(FACTOR * np.ceil(np.log(L_K))), L_K)
    u = min(int(FACTOR * np.ceil(np.log(L))), L)
    assert U_part == u

    idx_sample = jax.random.randint(
        jax.random.key(42), (L, U_part), 0, L_K)

    qf = queries.reshape(B, L, H * D)
    kf = keys.reshape(B, L, H * D)
    vf = values.reshape(B, L, H * D)

    blk_q, blk_c = 512, 256
    hp = 128 // D
    out = pl.pallas_call(
        functools.partial(_head_kernel, u=u, blk_q=blk_q, blk_c=blk_c,
                          d_head=D),
        grid=(B, H // hp),
        in_specs=[
            pl.BlockSpec((L, U_part), lambda b, h: (0, 0)),
            pl.BlockSpec((None, L, hp * D), lambda b, h: (b, 0, h)),
            pl.BlockSpec((None, L, hp * D), lambda b, h: (b, 0, h)),
            pl.BlockSpec((None, L, hp * D), lambda b, h: (b, 0, h)),
        ],
        out_specs=pl.BlockSpec((None, L, hp * D), lambda b, h: (b, 0, h)),
        out_shape=jax.ShapeDtypeStruct((B, L, H * D), jnp.float32),
        scratch_shapes=[
            pltpu.VMEM((L, L_K), jnp.float32),
            pltpu.VMEM((1, L), jnp.float32),
            pltpu.VMEM((u, D), jnp.float32),
            pltpu.VMEM((u, D), jnp.float32),
            pltpu.SMEM((u,), jnp.int32),
        ],
    )(idx_sample, qf, kf, vf)

    return out.reshape(B, L, H, D)
